# Initial kernel scaffold; baseline (speedup 1.0000x reference)
import jax
import jax.numpy as jnp
from jax.experimental import pallas as pl


def kernel(x, cu_seqlens, emb_tables, lin_W, lin_b):
    raise NotImplementedError("write your pallas kernel here")


# speedup vs baseline: 2.5228x; 2.5228x over previous
import functools

import jax
import jax.numpy as jnp
from jax import lax
from jax.experimental import pallas as pl
from jax.experimental.pallas import tpu as pltpu
from jax.experimental.pallas import tpu_sc as plsc

_CAT_IDX = (2, 4, 6, 8, 10, 12, 14, 16, 18, 20)
_CONT_IDX = tuple(n for n in range(21) if n not in _CAT_IDX)
_N = 21
_H = 16
_TOK = 16384
_NC = 2
_NS = 16
_NW = _NC * _NS
_TPW = _TOK // _NW
_C = 128
_NCH = _TPW // _C


def _sc_body(x_hbm, emb_hbm, w_hbm, b_hbm, out_hbm, x_v, o_v, emb_v, w_v, b_v):
    cid = lax.axis_index("c")
    sid = lax.axis_index("s")
    wid = sid * _NC + cid
    base = wid * _TPW

    pltpu.sync_copy(emb_hbm, emb_v)
    pltpu.sync_copy(w_hbm, w_v)
    pltpu.sync_copy(b_hbm, b_v)

    def chunk(ci, _):
        tok0 = base + ci * _C
        pltpu.sync_copy(x_hbm.at[pl.ds(tok0 * _N, _C * _N)], x_v)

        def per_tok(t, _):
            t_idx = jnp.broadcast_to(t * _N, (_H,)).astype(jnp.int32)
            o_base = t * (_N * _H)
            for n in range(_N):
                xv = plsc.load_gather(x_v, [t_idx + n])
                if n in _CAT_IDX:
                    k = _CAT_IDX.index(n)
                    e0 = emb_v[pl.ds(k * 2 * _H, _H)]
                    e1 = emb_v[pl.ds(k * 2 * _H + _H, _H)]
                    r = jnp.where(xv >= 1.0, e1, e0)
                else:
                    k = _CONT_IDX.index(n)
                    r = xv * w_v[pl.ds(k * _H, _H)] + b_v[pl.ds(k * _H, _H)]
                o_v[pl.ds(o_base + n * _H, _H)] = r
            return 0

        lax.fori_loop(0, _C, per_tok, 0)
        pltpu.sync_copy(o_v, out_hbm.at[pl.ds(tok0 * _N * _H, _C * _N * _H)])
        return 0

    lax.fori_loop(0, _NCH, chunk, 0)


@jax.jit
def _run(x, emb_tables, lin_W, lin_b):
    mesh = plsc.VectorSubcoreMesh(core_axis_name="c", subcore_axis_name="s")
    f = pl.kernel(
        _sc_body,
        out_type=jax.ShapeDtypeStruct((_TOK * _N * _H,), jnp.float32),
        mesh=mesh,
        compiler_params=pltpu.CompilerParams(needs_layout_passes=False),
        scratch_types=[
            pltpu.VMEM((_C * _N,), jnp.float32),
            pltpu.VMEM((_C * _N * _H,), jnp.float32),
            pltpu.VMEM((len(_CAT_IDX) * 2 * _H,), jnp.float32),
            pltpu.VMEM((len(_CONT_IDX) * _H,), jnp.float32),
            pltpu.VMEM((len(_CONT_IDX) * _H,), jnp.float32),
        ],
    )
    out = f(x.reshape(-1), emb_tables.reshape(-1), lin_W.reshape(-1),
            lin_b.reshape(-1))
    return out.reshape(_TOK, _N, _H)


def kernel(x, cu_seqlens, emb_tables, lin_W, lin_b):
    del cu_seqlens
    return _run(x.astype(jnp.float32), emb_tables, lin_W, lin_b)


# speedup vs baseline: 3.0499x; 1.2089x over previous
import functools

import jax
import jax.numpy as jnp
from jax import lax
from jax.experimental import pallas as pl
from jax.experimental.pallas import tpu as pltpu
from jax.experimental.pallas import tpu_sc as plsc

_CAT_IDX = (2, 4, 6, 8, 10, 12, 14, 16, 18, 20)
_CONT_IDX = tuple(n for n in range(21) if n not in _CAT_IDX)
_N = 21
_H = 16
_TOK = 16384
_NC = 2
_NS = 16
_NW = _NC * _NS
_TPW = _TOK // _NW
_C = 128
_NCH = _TPW // _C


def _sc_body(x_hbm, emb_hbm, w_hbm, b_hbm, out_hbm, x_v, o_v, emb_v, w_v, b_v):
    cid = lax.axis_index("c")
    sid = lax.axis_index("s")
    wid = sid * _NC + cid
    base = wid * _TPW

    pltpu.sync_copy(emb_hbm, emb_v)
    pltpu.sync_copy(w_hbm, w_v)
    pltpu.sync_copy(b_hbm, b_v)

    cat_rows = [
        (emb_v[pl.ds(k * 2 * _H, _H)], emb_v[pl.ds(k * 2 * _H + _H, _H)])
        for k in range(len(_CAT_IDX))
    ]
    cont_rows = [
        (w_v[pl.ds(k * _H, _H)], b_v[pl.ds(k * _H, _H)])
        for k in range(len(_CONT_IDX))
    ]
    lanes = jnp.arange(_H, dtype=jnp.int32)

    def chunk(ci, _):
        tok0 = base + ci * _C
        pltpu.sync_copy(x_hbm.at[pl.ds(tok0 * _N, _C * _N)], x_v)

        def per_tok(t, _):
            off = t * _N
            o_base = t * (_N * _H)
            v0 = plsc.load_gather(x_v, [off + lanes])
            v1 = plsc.load_gather(x_v, [off + (_N - _H) + lanes])
            for n in range(_N):
                if n < _H:
                    xv = jnp.take_along_axis(
                        v0, jnp.full((_H,), n, jnp.int32), axis=0,
                        mode="promise_in_bounds")
                else:
                    xv = jnp.take_along_axis(
                        v1, jnp.full((_H,), n - (_N - _H), jnp.int32), axis=0,
                        mode="promise_in_bounds")
                if n in _CAT_IDX:
                    e0, e1 = cat_rows[_CAT_IDX.index(n)]
                    r = jnp.where(xv >= 1.0, e1, e0)
                else:
                    w, b = cont_rows[_CONT_IDX.index(n)]
                    r = xv * w + b
                o_v[pl.ds(o_base + n * _H, _H)] = r
            return 0

        lax.fori_loop(0, _C, per_tok, 0)
        pltpu.sync_copy(o_v, out_hbm.at[pl.ds(tok0 * _N * _H, _C * _N * _H)])
        return 0

    lax.fori_loop(0, _NCH, chunk, 0)


@jax.jit
def _run(x, emb_tables, lin_W, lin_b):
    mesh = plsc.VectorSubcoreMesh(core_axis_name="c", subcore_axis_name="s")
    f = pl.kernel(
        _sc_body,
        out_type=jax.ShapeDtypeStruct((_TOK * _N * _H,), jnp.float32),
        mesh=mesh,
        compiler_params=pltpu.CompilerParams(needs_layout_passes=False),
        scratch_types=[
            pltpu.VMEM((_C * _N,), jnp.float32),
            pltpu.VMEM((_C * _N * _H,), jnp.float32),
            pltpu.VMEM((len(_CAT_IDX) * 2 * _H,), jnp.float32),
            pltpu.VMEM((len(_CONT_IDX) * _H,), jnp.float32),
            pltpu.VMEM((len(_CONT_IDX) * _H,), jnp.float32),
        ],
    )
    out = f(x.reshape(-1), emb_tables.reshape(-1), lin_W.reshape(-1),
            lin_b.reshape(-1))
    return out.reshape(_TOK, _N, _H)


def kernel(x, cu_seqlens, emb_tables, lin_W, lin_b):
    del cu_seqlens
    return _run(x.astype(jnp.float32), emb_tables, lin_W, lin_b)


# speedup vs baseline: 14.0769x; 4.6155x over previous
import jax
import jax.numpy as jnp
from jax import lax
from jax.experimental import pallas as pl
from jax.experimental.pallas import tpu as pltpu
from jax.experimental.pallas import tpu_sc as plsc

_CAT_IDX = (2, 4, 6, 8, 10, 12, 14, 16, 18, 20)
_CONT_IDX = tuple(n for n in range(21) if n not in _CAT_IDX)
_N = 21
_H = 16
_L = 16
_TOK = 16384
_NC = 2
_NS = 16
_NW = _NC * _NS
_TPW = 1024
_NTG = _TPW // _L


def _sc_body(x_hbm, emb_hbm, w_hbm, b_hbm, out_hbm,
             x_v, o_v0, o_v1, emb_v, w_v, b_v, sem0, sem1):
    cid = lax.axis_index("c")
    sid = lax.axis_index("s")
    wid = sid * _NC + cid
    kk = wid & 1
    tb = wid >> 1
    k8 = kk * 8

    pltpu.sync_copy(emb_hbm, emb_v)
    pltpu.sync_copy(w_hbm, w_v)
    pltpu.sync_copy(b_hbm, b_v)
    pltpu.sync_copy(x_hbm.at[pl.ds(tb * (_TPW * _N), _TPW * _N)], x_v)

    lanes = jnp.arange(_L, dtype=jnp.int32)
    iota_n = lanes * _N

    o_bufs = (o_v0, o_v1)
    sems = (sem0, sem1)
    pending = [None, None]

    piece_w = (kk * 128 + tb * 8) * 1024

    def bcast(row, k):
        return jnp.take_along_axis(
            row, jnp.full((_L,), k8 + k, jnp.int32), axis=0,
            mode="promise_in_bounds")

    for n in range(_N):
        b = n & 1
        o_v = o_bufs[b]
        if pending[b] is not None:
            pending[b].wait()

        if n in _CAT_IDX:
            ci = _CAT_IDX.index(n)
            e0r = emb_v[pl.ds(ci * 2 * _H, _H)]
            e1r = emb_v[pl.ds(ci * 2 * _H + _H, _H)]
            e0b = [bcast(e0r, k) for k in range(8)]
            e1b = [bcast(e1r, k) for k in range(8)]
        else:
            li = _CONT_IDX.index(n)
            wb = [bcast(w_v[pl.ds(li * _H, _H)], k) for k in range(8)]
            bb = [bcast(b_v[pl.ds(li * _H, _H)], k) for k in range(8)]

        base_idx = iota_n + n
        is_cat = n in _CAT_IDX

        def per_tg(tg, _):
            xv = plsc.load_gather(x_v, [base_idx + tg * (_L * _N)])
            off = (tg >> 3) * 1024 + (tg & 7) * _L
            if is_cat:
                m = xv >= 1.0
                for k in range(8):
                    o_v[pl.ds(off + k * 128, _L)] = jnp.where(m, e1b[k], e0b[k])
            else:
                for k in range(8):
                    o_v[pl.ds(off + k * 128, _L)] = xv * wb[k] + bb[k]
            return 0

        lax.fori_loop(0, _NTG, per_tg, 0)

        dst = out_hbm.at[pl.ds(n * (_H * _TOK) + piece_w, 8 * _TPW)]
        pending[b] = pltpu.async_copy(o_v, dst, sems[b])

    pending[0].wait()
    pending[1].wait()


@jax.jit
def _run(x, emb_tables, lin_W, lin_b):
    mesh = plsc.VectorSubcoreMesh(core_axis_name="c", subcore_axis_name="s")
    f = pl.kernel(
        _sc_body,
        out_type=jax.ShapeDtypeStruct((_TOK * _N * _H,), jnp.float32),
        mesh=mesh,
        compiler_params=pltpu.CompilerParams(needs_layout_passes=False),
        scratch_types=[
            pltpu.VMEM((_TPW * _N,), jnp.float32),
            pltpu.VMEM((8 * _TPW,), jnp.float32),
            pltpu.VMEM((8 * _TPW,), jnp.float32),
            pltpu.VMEM((len(_CAT_IDX) * 2 * _H,), jnp.float32),
            pltpu.VMEM((len(_CONT_IDX) * _H,), jnp.float32),
            pltpu.VMEM((len(_CONT_IDX) * _H,), jnp.float32),
            pltpu.SemaphoreType.DMA,
            pltpu.SemaphoreType.DMA,
        ],
    )
    out = f(x.reshape(-1), emb_tables.reshape(-1), lin_W.reshape(-1),
            lin_b.reshape(-1))
    buf = out.reshape(_N, 2, _TOK // 128, 8, 128)
    return buf.transpose(2, 4, 0, 1, 3).reshape(_TOK, _N, _H)


def kernel(x, cu_seqlens, emb_tables, lin_W, lin_b):
    del cu_seqlens
    return _run(x.astype(jnp.float32), emb_tables, lin_W, lin_b)


# speedup vs baseline: 14.5603x; 1.0343x over previous
import jax
import jax.numpy as jnp
from jax import lax
from jax.experimental import pallas as pl
from jax.experimental.pallas import tpu as pltpu
from jax.experimental.pallas import tpu_sc as plsc

_CAT_IDX = (2, 4, 6, 8, 10, 12, 14, 16, 18, 20)
_CONT_IDX = tuple(n for n in range(21) if n not in _CAT_IDX)
_N = 21
_H = 16
_L = 16
_TOK = 16384
_NC = 2
_NS = 16
_NW = _NC * _NS
_TPW = 1024
_NTG = _TPW // _L
_EMB0 = 0
_W0 = len(_CAT_IDX) * 2 * _H
_B0 = _W0 + len(_CONT_IDX) * _H
_WLEN = _B0 + len(_CONT_IDX) * _H


def _sc_body(x_hbm, wts_hbm, out_hbm, x_v, o_v0, o_v1, wts_v, sem0, sem1):
    cid = lax.axis_index("c")
    sid = lax.axis_index("s")
    wid = sid * _NC + cid
    kk = wid & 1
    tb = wid >> 1
    k8 = kk * 8

    pltpu.sync_copy(wts_hbm, wts_v)
    pltpu.sync_copy(x_hbm.at[pl.ds(tb * (_TPW * _N), _TPW * _N)], x_v)

    lanes = jnp.arange(_L, dtype=jnp.int32)

    o_bufs = (o_v0, o_v1)
    sems = (sem0, sem1)
    pending = [None, None]

    piece_w = (kk * 128 + tb * 8) * 1024

    def bcast(off, k):
        row = wts_v[pl.ds(off, _H)]
        return jnp.take_along_axis(
            row, jnp.full((_L,), k8 + k, jnp.int32), axis=0,
            mode="promise_in_bounds")

    for n in range(_N):
        b = n & 1
        o_v = o_bufs[b]
        if pending[b] is not None:
            pending[b].wait()

        is_cat = n in _CAT_IDX
        if is_cat:
            ci = _CAT_IDX.index(n)
            e0b = [bcast(_EMB0 + ci * 2 * _H, k) for k in range(8)]
            e1b = [bcast(_EMB0 + ci * 2 * _H + _H, k) for k in range(8)]
        else:
            li = _CONT_IDX.index(n)
            wb = [bcast(_W0 + li * _H, k) for k in range(8)]
            bb = [bcast(_B0 + li * _H, k) for k in range(8)]

        base_idx = lanes * _N + n

        def per_tg(tg, _):
            xv = plsc.load_gather(x_v, [base_idx + tg * (_L * _N)])
            off = (tg >> 3) * 1024 + (tg & 7) * _L
            if is_cat:
                m = xv >= 1.0
                for k in range(8):
                    o_v[pl.ds(off + k * 128, _L)] = jnp.where(m, e1b[k], e0b[k])
            else:
                for k in range(8):
                    o_v[pl.ds(off + k * 128, _L)] = xv * wb[k] + bb[k]
            return 0

        lax.fori_loop(0, _NTG, per_tg, 0, unroll=2)

        dst = out_hbm.at[pl.ds(n * (_H * _TOK) + piece_w, 8 * _TPW)]
        pending[b] = pltpu.async_copy(o_v, dst, sems[b])

    pending[0].wait()
    pending[1].wait()


@jax.jit
def _run(x, emb_tables, lin_W, lin_b):
    wts = jnp.concatenate(
        [emb_tables.reshape(-1), lin_W.reshape(-1), lin_b.reshape(-1)])
    mesh = plsc.VectorSubcoreMesh(core_axis_name="c", subcore_axis_name="s")
    f = pl.kernel(
        _sc_body,
        out_type=jax.ShapeDtypeStruct((_TOK * _N * _H,), jnp.float32),
        mesh=mesh,
        compiler_params=pltpu.CompilerParams(needs_layout_passes=False),
        scratch_types=[
            pltpu.VMEM((_TPW * _N,), jnp.float32),
            pltpu.VMEM((8 * _TPW,), jnp.float32),
            pltpu.VMEM((8 * _TPW,), jnp.float32),
            pltpu.VMEM((_WLEN,), jnp.float32),
            pltpu.SemaphoreType.DMA,
            pltpu.SemaphoreType.DMA,
        ],
    )
    out = f(x.reshape(-1), wts)
    buf = out.reshape(_N, 2, _TOK // 128, 8, 128)
    return buf.transpose(2, 4, 0, 1, 3).reshape(_TOK, _N, _H)


def kernel(x, cu_seqlens, emb_tables, lin_W, lin_b):
    del cu_seqlens
    return _run(x.astype(jnp.float32), emb_tables, lin_W, lin_b)
